# baseline (speedup 1.0000x reference)
import math

import jax
import jax.numpy as jnp
from jax.experimental import pallas as pl
from jax.experimental.pallas import tpu as pltpu

_DT = jnp.bfloat16

_TRANS_A = (((0,), (0,)), ((), ()))


def _lap_kernel(row_ref, col_ref, lap_ref):
    e = row_ref.shape[0]
    n = lap_ref.shape[0]
    ids = jax.lax.broadcasted_iota(jnp.int32, (e, n), 1)
    er = (ids == row_ref[...]).astype(_DT)
    ec = (ids == col_ref[...]).astype(_DT)
    a = jax.lax.dot_general(er, ec, _TRANS_A,
                            preferred_element_type=jnp.float32)
    deg = jnp.sum(a, axis=1, keepdims=True)
    rows = jax.lax.broadcasted_iota(jnp.int32, (n, n), 0)
    cols = jax.lax.broadcasted_iota(jnp.int32, (n, n), 1)
    lap_ref[...] = jnp.where(rows == cols, deg - a, -a)


def _compose_g_kernel(c_ref, lap_ref, g_ref):
    i = pl.program_id(0)
    h = c_ref[i, 0]
    alpha = c_ref[i, 1]
    lap = lap_ref[...]
    n = lap.shape[0]
    rows = jax.lax.broadcasted_iota(jnp.int32, (n, n), 0)
    cols = jax.lax.broadcasted_iota(jnp.int32, (n, n), 1)
    eye = rows == cols

    off = jnp.where(eye, 0.0, lap).astype(_DT)
    ld = jnp.sum(jnp.where(eye, lap, 0.0), axis=1, keepdims=True) - alpha
    hld = h * ld
    denom = 1.0 / (hld * hld + 1.0)
    tlr = hld * denom
    tli = -denom
    dr = (-h) * tlr
    di = (-h) * tli

    hlm = h * jnp.where(eye, lap - alpha, lap)
    br = tlr * hlm + jnp.where(eye, tli, 0.0)
    bi = tli * hlm - jnp.where(eye, tlr, 0.0)

    def jmul(ur, ui):
        sr = (dr * ur - di * ui).astype(_DT)
        si = (dr * ui + di * ur).astype(_DT)
        return (jax.lax.dot_general(off, sr, _TRANS_A,
                                    preferred_element_type=jnp.float32),
                jax.lax.dot_general(off, si, _TRANS_A,
                                    preferred_element_type=jnp.float32))

    jbr, jbi = jmul(br, bi)
    mr, mi = jmul(jbr + br, jbi + bi)
    mr = mr + br
    mi = mi + bi

    mrl = mr.astype(_DT)
    mil = mi.astype(_DT)
    msl = (mr + mi).astype(_DT)
    t1 = jnp.dot(mrl, mrl, preferred_element_type=jnp.float32)
    t2 = jnp.dot(mil, mil, preferred_element_type=jnp.float32)
    t3 = jnp.dot(msl, msl, preferred_element_type=jnp.float32)
    m2r = t1 - t2
    m2i = t3 - t1 - t2

    g = (2.0 * (c_ref[i, 3] * mr - c_ref[i, 4] * mi)
         + 2.0 * (c_ref[i, 5] * m2r - c_ref[i, 6] * m2i))
    g_ref[0] = (g + jnp.where(eye, c_ref[i, 2], 0.0)).astype(g_ref.dtype)


def _apply_convs_kernel(g_ref, w_ref, x_ref, out_ref, acc_ref):
    x = x_ref[...].astype(_DT)
    hid = jnp.dot(g_ref[0], x, preferred_element_type=jnp.float32)
    hid = jnp.maximum(hid, 0.0).astype(_DT)
    o = jnp.dot(g_ref[1], hid, preferred_element_type=jnp.float32)
    o = jnp.maximum(o, 0.0)
    out_ref[...] = o

    @pl.when(pl.program_id(0) == 0)
    def _():
        acc_ref[...] = jnp.zeros_like(acc_ref)

    acc_ref[...] += jnp.dot(o, w_ref[...],
                            preferred_element_type=jnp.float32)


def kernel(x, edge_index, batch,
           conv0_h, conv0_alpha, conv0_c0, conv0_cjr, conv0_cji,
           conv1_h, conv1_alpha, conv1_c0, conv1_cjr, conv1_cji,
           pool_w, lin_w, lin_b):
    del batch
    n, f = x.shape
    e = edge_index.shape[1]

    lap = pl.pallas_call(
        _lap_kernel,
        out_shape=jax.ShapeDtypeStruct((n, n), jnp.float32),
        in_specs=[pl.BlockSpec((e, 1), lambda: (0, 0)),
                  pl.BlockSpec((e, 1), lambda: (0, 0))],
        out_specs=pl.BlockSpec((n, n), lambda: (0, 0)),
    )(edge_index[0].reshape(e, 1), edge_index[1].reshape(e, 1))

    cvec = jnp.stack([
        jnp.stack([conv0_h, conv0_alpha, conv0_c0, conv0_cjr[0],
                   conv0_cji[0], conv0_cjr[1], conv0_cji[1]]),
        jnp.stack([conv1_h, conv1_alpha, conv1_c0, conv1_cjr[0],
                   conv1_cji[0], conv1_cjr[1], conv1_cji[1]]),
    ]).astype(jnp.float32)

    g = pl.pallas_call(
        _compose_g_kernel,
        out_shape=jax.ShapeDtypeStruct((2, n, n), _DT),
        grid=(2,),
        in_specs=[pl.BlockSpec(memory_space=pltpu.MemorySpace.SMEM),
                  pl.BlockSpec((n, n), lambda i: (0, 0))],
        out_specs=pl.BlockSpec((1, n, n), lambda i: (i, 0, 0)),
        compiler_params=pltpu.CompilerParams(
            dimension_semantics=("arbitrary",)),
    )(cvec, lap)

    tf = min(512, f)
    out, sacc = pl.pallas_call(
        _apply_convs_kernel,
        out_shape=[jax.ShapeDtypeStruct((n, f), jnp.float32),
                   jax.ShapeDtypeStruct((n, 1), jnp.float32)],
        grid=(f // tf,),
        in_specs=[pl.BlockSpec((2, n, n), lambda i: (0, 0, 0)),
                  pl.BlockSpec((tf, 1), lambda i: (i, 0)),
                  pl.BlockSpec((n, tf), lambda i: (0, i))],
        out_specs=[pl.BlockSpec((n, tf), lambda i: (0, i)),
                   pl.BlockSpec((n, 1), lambda i: (0, 0))],
        compiler_params=pltpu.CompilerParams(
            dimension_semantics=("arbitrary",)),
    )(g, pool_w.reshape(f, 1), x)

    score = jnp.tanh(sacc[:, 0] / jnp.linalg.norm(pool_w))
    kk = int(math.ceil(0.9 * n))
    vals, perm = jax.lax.top_k(score, kk)
    wv = jnp.zeros((n,), jnp.float32).at[perm].set(vals)
    pooled = jnp.dot(wv, out) / kk
    return jnp.dot(pooled[None, :], lin_w.T) + lin_b


# speedup vs baseline: 1.0548x; 1.0548x over previous
import functools
import math

import jax
import jax.numpy as jnp
from jax.experimental import pallas as pl
from jax.experimental.pallas import tpu as pltpu

_DT = jnp.bfloat16

_TRANS_A = (((0,), (0,)), ((), ()))
_TRANS_B = (((1,), (1,)), ((), ()))


def _lap_kernel(row_ref, col_ref, lap_ref):
    e = row_ref.shape[0]
    n = lap_ref.shape[0]
    ids = jax.lax.broadcasted_iota(jnp.int32, (e, n), 1)
    er = (ids == row_ref[...]).astype(_DT)
    ec = (ids == col_ref[...]).astype(_DT)
    a = jax.lax.dot_general(er, ec, _TRANS_A,
                            preferred_element_type=jnp.float32)
    deg = jnp.sum(a, axis=1, keepdims=True)
    rows = jax.lax.broadcasted_iota(jnp.int32, (n, n), 0)
    cols = jax.lax.broadcasted_iota(jnp.int32, (n, n), 1)
    lap_ref[...] = jnp.where(rows == cols, deg - a, -a)


def _compose_g_kernel(c_ref, lap_ref, g_ref):
    i = pl.program_id(0)
    h = c_ref[i, 0]
    alpha = c_ref[i, 1]
    lap = lap_ref[...]
    n = lap.shape[0]
    rows = jax.lax.broadcasted_iota(jnp.int32, (n, n), 0)
    cols = jax.lax.broadcasted_iota(jnp.int32, (n, n), 1)
    eye = rows == cols

    off = jnp.where(eye, 0.0, lap).astype(_DT)
    ld = jnp.sum(jnp.where(eye, lap, 0.0), axis=1, keepdims=True) - alpha
    hld = h * ld
    denom = 1.0 / (hld * hld + 1.0)
    tlr = hld * denom
    tli = -denom
    dr = (-h) * tlr
    di = (-h) * tli

    hlm = h * jnp.where(eye, lap - alpha, lap)
    br = tlr * hlm + jnp.where(eye, tli, 0.0)
    bi = tli * hlm - jnp.where(eye, tlr, 0.0)

    def jmul(ur, ui):
        sr = (dr * ur - di * ui).astype(_DT)
        si = (dr * ui + di * ur).astype(_DT)
        return (jax.lax.dot_general(off, sr, _TRANS_A,
                                    preferred_element_type=jnp.float32),
                jax.lax.dot_general(off, si, _TRANS_A,
                                    preferred_element_type=jnp.float32))

    jbr, jbi = jmul(br, bi)
    mr, mi = jmul(jbr + br, jbi + bi)
    mr = mr + br
    mi = mi + bi

    mrl = mr.astype(_DT)
    mil = mi.astype(_DT)
    msl = (mr + mi).astype(_DT)
    t1 = jnp.dot(mrl, mrl, preferred_element_type=jnp.float32)
    t2 = jnp.dot(mil, mil, preferred_element_type=jnp.float32)
    t3 = jnp.dot(msl, msl, preferred_element_type=jnp.float32)
    m2r = t1 - t2
    m2i = t3 - t1 - t2

    g = (2.0 * (c_ref[i, 3] * mr - c_ref[i, 4] * mi)
         + 2.0 * (c_ref[i, 5] * m2r - c_ref[i, 6] * m2i))
    g_ref[0] = (g + jnp.where(eye, c_ref[i, 2], 0.0)).astype(g_ref.dtype)


def _apply_convs_kernel(g_ref, w_ref, x_ref, out_ref, acc_ref):
    x = x_ref[...].astype(_DT)
    hid = jnp.dot(g_ref[0], x, preferred_element_type=jnp.float32)
    hid = jnp.maximum(hid, 0.0).astype(_DT)
    o = jnp.dot(g_ref[1], hid, preferred_element_type=jnp.float32)
    o = jnp.maximum(o, 0.0)
    out_ref[...] = o.astype(out_ref.dtype)

    @pl.when(pl.program_id(0) == 0)
    def _():
        acc_ref[...] = jnp.zeros_like(acc_ref)

    acc_ref[...] += jax.lax.dot_general(
        w_ref[...], o, (((0,), (1,)), ((), ())),
        preferred_element_type=jnp.float32)


def _epilogue_kernel(sacc_ref, pw_ref, lb_ref, out_ref, lw_ref, res_ref,
                     wv_ref, *, kk):
    i = pl.program_id(0)
    n = wv_ref.shape[1]

    @pl.when(i == 0)
    def _():
        pw = pw_ref[...]
        inv_norm = 1.0 / jnp.sqrt(jnp.sum(pw * pw))
        score = jnp.tanh(sacc_ref[...] * inv_norm)
        b = pltpu.bitcast(score, jnp.int32)
        key = jnp.where(b < 0, jnp.int32(-2147483648) - b, b)

        top = jnp.int32(0x3F800000)

        def cnt_ge(t):
            return jnp.sum((key >= t).astype(jnp.int32))

        def body(_, carry):
            lo, hi = carry
            mid = lo + (hi - lo + 1) // 2
            go = cnt_ge(mid) >= kk
            return (jnp.where(go, mid, lo), jnp.where(go, hi, mid - 1))

        lo, _ = jax.lax.fori_loop(0, 32, body, (-top, top))

        c_gt = jnp.sum((key > lo).astype(jnp.int32))
        tie = key == lo
        ii = jax.lax.broadcasted_iota(jnp.int32, (n, n), 0)
        jj = jax.lax.broadcasted_iota(jnp.int32, (n, n), 1)
        tri = (ii <= jj).astype(_DT)
        pref = jnp.dot(tie.astype(_DT)[...], tri,
                       preferred_element_type=jnp.float32)
        sel = (key > lo) | (tie & (pref <= (kk - c_gt).astype(jnp.float32)))
        wv_ref[...] = jnp.where(sel, score, 0.0)

    vt = jnp.dot(wv_ref[...].astype(_DT), out_ref[...],
                 preferred_element_type=jnp.float32) * (1.0 / kk)
    ft = jax.lax.dot_general(vt.astype(_DT), lw_ref[...].astype(_DT),
                             _TRANS_B, preferred_element_type=jnp.float32)

    @pl.when(i == 0)
    def _():
        res_ref[...] = lb_ref[...]

    res_ref[...] += ft


def kernel(x, edge_index, batch,
           conv0_h, conv0_alpha, conv0_c0, conv0_cjr, conv0_cji,
           conv1_h, conv1_alpha, conv1_c0, conv1_cjr, conv1_cji,
           pool_w, lin_w, lin_b):
    del batch
    n, f = x.shape
    e = edge_index.shape[1]
    nout = lin_w.shape[0]

    lap = pl.pallas_call(
        _lap_kernel,
        out_shape=jax.ShapeDtypeStruct((n, n), jnp.float32),
        in_specs=[pl.BlockSpec((e, 1), lambda: (0, 0)),
                  pl.BlockSpec((e, 1), lambda: (0, 0))],
        out_specs=pl.BlockSpec((n, n), lambda: (0, 0)),
    )(edge_index[0].reshape(e, 1), edge_index[1].reshape(e, 1))

    cvec = jnp.stack([
        jnp.stack([conv0_h, conv0_alpha, conv0_c0, conv0_cjr[0],
                   conv0_cji[0], conv0_cjr[1], conv0_cji[1]]),
        jnp.stack([conv1_h, conv1_alpha, conv1_c0, conv1_cjr[0],
                   conv1_cji[0], conv1_cjr[1], conv1_cji[1]]),
    ]).astype(jnp.float32)

    g = pl.pallas_call(
        _compose_g_kernel,
        out_shape=jax.ShapeDtypeStruct((2, n, n), _DT),
        grid=(2,),
        in_specs=[pl.BlockSpec(memory_space=pltpu.MemorySpace.SMEM),
                  pl.BlockSpec((n, n), lambda i: (0, 0))],
        out_specs=pl.BlockSpec((1, n, n), lambda i: (i, 0, 0)),
        compiler_params=pltpu.CompilerParams(
            dimension_semantics=("arbitrary",)),
    )(cvec, lap)

    tf = min(512, f)
    out, sacc = pl.pallas_call(
        _apply_convs_kernel,
        out_shape=[jax.ShapeDtypeStruct((n, f), _DT),
                   jax.ShapeDtypeStruct((1, n), jnp.float32)],
        grid=(f // tf,),
        in_specs=[pl.BlockSpec((2, n, n), lambda i: (0, 0, 0)),
                  pl.BlockSpec((tf, 1), lambda i: (i, 0)),
                  pl.BlockSpec((n, tf), lambda i: (0, i))],
        out_specs=[pl.BlockSpec((n, tf), lambda i: (0, i)),
                   pl.BlockSpec((1, n), lambda i: (0, 0))],
        compiler_params=pltpu.CompilerParams(
            dimension_semantics=("arbitrary",)),
    )(g, pool_w.reshape(f, 1), x)

    kk = int(math.ceil(0.9 * n))
    res = pl.pallas_call(
        functools.partial(_epilogue_kernel, kk=kk),
        out_shape=jax.ShapeDtypeStruct((1, nout), jnp.float32),
        grid=(f // tf,),
        in_specs=[pl.BlockSpec((1, n), lambda i: (0, 0)),
                  pl.BlockSpec((1, f), lambda i: (0, 0)),
                  pl.BlockSpec((1, nout), lambda i: (0, 0)),
                  pl.BlockSpec((n, tf), lambda i: (0, i)),
                  pl.BlockSpec((nout, tf), lambda i: (0, i))],
        out_specs=pl.BlockSpec((1, nout), lambda i: (0, 0)),
        scratch_shapes=[pltpu.VMEM((1, n), jnp.float32)],
        compiler_params=pltpu.CompilerParams(
            dimension_semantics=("arbitrary",)),
    )(sacc, pool_w.reshape(1, f), lin_b.reshape(1, nout), out, lin_w)
    return res
